# baseline (speedup 1.0000x reference)
import jax
import jax.numpy as jnp
from jax import lax
from jax.experimental import pallas as pl
from jax.experimental.pallas import tpu as pltpu
from jax.experimental.pallas import tpu_sc as plsc

_V = 4096
_D = 32
_NC = 2
_NS = 16
_NW = _NC * _NS


def _sc_gather_body(bias4_hbm, bt4_hbm, eg4_hbm, idg4_hbm,
                    bout_hbm, hout_hbm,
                    eg4_v, idg4_v, brows_v, hrows_v, sem_b, sem_h):
    rpw = eg4_v.shape[0]
    wid = lax.axis_index("s") * _NC + lax.axis_index("c")
    base = wid * rpw
    pltpu.sync_copy(eg4_hbm.at[pl.ds(base, rpw)], eg4_v)
    pltpu.sync_copy(idg4_hbm.at[pl.ds(base, rpw)], idg4_v)
    cb = pltpu.async_copy(bias4_hbm.at[eg4_v], brows_v, sem_b)
    ch = pltpu.async_copy(bt4_hbm.at[idg4_v], hrows_v, sem_h)
    cb.wait()
    ch.wait()
    pltpu.sync_copy(brows_v, bout_hbm.at[pl.ds(base, rpw)])
    pltpu.sync_copy(hrows_v, hout_hbm.at[pl.ds(base, rpw)])


def _extract32(rows128, sub):
    lane_grp = lax.broadcasted_iota(jnp.int32, (1, 128), 1) // _D
    masked = jnp.where(lane_grp == sub, rows128, 0.0)
    return (masked[:, 0:32] + masked[:, 32:64]
            + masked[:, 64:96] + masked[:, 96:128])


_NBUF = 16


def _tc_scan_body(W_hbm, eidx_s, bg_ref, hg_ref, eidx_ref, ids_ref, pe_ref,
                  a_ref, bt_ref, sc_ref, logits_ref, probs_ref,
                  c_ref, wring, sems):
    L = hg_ref.shape[0]
    nsteps = L - 1
    gate = sc_ref[0]
    pe_scale = sc_ref[1]

    for k in range(_NBUF):
        pltpu.make_async_copy(W_hbm.at[eidx_s[k]],
                              wring.at[pl.ds(k * _D, _D), :],
                              sems.at[k]).start()

    be = _extract32(bg_ref[...], eidx_ref[...] & 3)
    hrow = _extract32(hg_ref[...], ids_ref[...] & 3)
    h = (hrow + pe_scale * pe_ref[...]) * a_ref[...]
    c_ref[pl.ds(0, nsteps), :] = be[0:nsteps, :] + h[1:, :]
    e0 = h[0:1, :]

    def step(t, e):
        slot = lax.rem(t, _NBUF)
        off = pl.multiple_of(slot * _D, _D)
        pltpu.make_async_copy(W_hbm.at[0], wring.at[pl.ds(off, _D), :],
                              sems.at[slot]).wait()
        Wt = wring[pl.ds(off, _D), :]
        We = lax.dot_general(e, Wt, (((1,), (1,)), ((), ())),
                             preferred_element_type=jnp.float32)
        e_new = jnp.tanh(We + c_ref[pl.ds(t, 1), :])
        e2 = gate * e_new + (1.0 - gate) * e
        nt = t + _NBUF

        @pl.when(nt < nsteps)
        def _refill():
            pltpu.make_async_copy(W_hbm.at[eidx_s[nt]],
                                  wring.at[pl.ds(off, _D), :],
                                  sems.at[slot]).start()

        return e2

    e = lax.fori_loop(0, nsteps, step, e0)
    logits = lax.dot_general(e, bt_ref[...], (((1,), (1,)), ((), ())),
                             preferred_element_type=jnp.float32)
    logits_ref[...] = logits
    m = jnp.max(logits, axis=1, keepdims=True)
    ex = jnp.exp(logits - m)
    probs_ref[...] = ex / jnp.sum(ex, axis=1, keepdims=True)


def kernel(ids, eids, bias_table, W, bias, W_shared, bias_shared, a, gate,
           pe_scale, PE_cache):
    L = ids.shape[0]
    E = W.shape[0]
    rpw = L // _NW
    bias4 = bias.reshape(E // 4, 4 * _D)
    bt4 = bias_table.reshape(_V // 4, 4 * _D)
    eidx = jnp.concatenate([eids, eids[:1]]).astype(jnp.int32)
    ids32 = ids.astype(jnp.int32)
    eg4 = eidx // 4
    idg4 = ids32 // 4

    sc_gather = pl.kernel(
        _sc_gather_body,
        out_type=[jax.ShapeDtypeStruct((L, 4 * _D), jnp.float32),
                  jax.ShapeDtypeStruct((L, 4 * _D), jnp.float32)],
        mesh=plsc.VectorSubcoreMesh(core_axis_name="c", subcore_axis_name="s"),
        scratch_types=[pltpu.VMEM((rpw,), jnp.int32),
                       pltpu.VMEM((rpw,), jnp.int32),
                       pltpu.VMEM((rpw, 4 * _D), jnp.float32),
                       pltpu.VMEM((rpw, 4 * _D), jnp.float32),
                       pltpu.SemaphoreType.DMA,
                       pltpu.SemaphoreType.DMA],
    )
    bg, hg = sc_gather(bias4, bt4, eg4, idg4)

    sc2 = jnp.stack([jnp.asarray(gate, jnp.float32),
                     jnp.asarray(pe_scale, jnp.float32)])
    a2d = a[0].astype(jnp.float32)
    eidx2 = eidx.reshape(L, 1)
    ids2 = ids32.reshape(L, 1)

    logits2, probs2 = pl.pallas_call(
        _tc_scan_body,
        out_shape=[jax.ShapeDtypeStruct((1, _V), jnp.float32),
                   jax.ShapeDtypeStruct((1, _V), jnp.float32)],
        in_specs=[pl.BlockSpec(memory_space=pl.ANY),
                  pl.BlockSpec(memory_space=pltpu.SMEM)]
        + [pl.BlockSpec(memory_space=pltpu.VMEM)] * 7
        + [pl.BlockSpec(memory_space=pltpu.SMEM)],
        out_specs=[pl.BlockSpec(memory_space=pltpu.VMEM)] * 2,
        scratch_shapes=[pltpu.VMEM((L, _D), jnp.float32),
                        pltpu.VMEM((_NBUF * _D, _D), jnp.float32),
                        pltpu.SemaphoreType.DMA((_NBUF,))],
    )(W, eidx, bg, hg, eidx2, ids2, PE_cache, a2d, bias_table, sc2)
    return logits2[0], probs2[0]


# speedup vs baseline: 2.5525x; 2.5525x over previous
import jax
import jax.numpy as jnp
from jax import lax
from jax.experimental import pallas as pl
from jax.experimental.pallas import tpu as pltpu
from jax.experimental.pallas import tpu_sc as plsc

_V = 4096
_D = 32
_NC = 2
_NS = 16
_NW = _NC * _NS


def _sc_gather_body(Wf_hbm, bias4_hbm, bt4_hbm, eidx_hbm, eg4_hbm, idg4_hbm,
                    Wout_hbm, bout_hbm, hout_hbm,
                    eidx_v, eg4_v, idg4_v, wrows_v, brows_v, hrows_v,
                    sem_w, sem_b, sem_h):
    rpw = eidx_v.shape[0]
    wid = lax.axis_index("s") * _NC + lax.axis_index("c")
    base = wid * rpw
    pltpu.sync_copy(eidx_hbm.at[pl.ds(base, rpw)], eidx_v)
    pltpu.sync_copy(eg4_hbm.at[pl.ds(base, rpw)], eg4_v)
    pltpu.sync_copy(idg4_hbm.at[pl.ds(base, rpw)], idg4_v)
    cw = pltpu.async_copy(Wf_hbm.at[eidx_v], wrows_v, sem_w)
    cb = pltpu.async_copy(bias4_hbm.at[eg4_v], brows_v, sem_b)
    ch = pltpu.async_copy(bt4_hbm.at[idg4_v], hrows_v, sem_h)
    cw.wait()
    cb.wait()
    ch.wait()
    pltpu.sync_copy(wrows_v, Wout_hbm.at[pl.ds(base, rpw)])
    pltpu.sync_copy(brows_v, bout_hbm.at[pl.ds(base, rpw)])
    pltpu.sync_copy(hrows_v, hout_hbm.at[pl.ds(base, rpw)])


def _extract32(rows128, sub):
    lane_grp = lax.broadcasted_iota(jnp.int32, (1, 128), 1) // _D
    masked = jnp.where(lane_grp == sub, rows128, 0.0)
    return (masked[:, 0:32] + masked[:, 32:64]
            + masked[:, 64:96] + masked[:, 96:128])


def _tc_scan_body(W3_ref, bg_ref, hg_ref, eidx_ref, ids_ref, pe_ref, a_ref,
                  bt_ref, sc_ref, logits_ref, probs_ref,
                  c_ref, ccol_ref, w3t_ref):
    L = hg_ref.shape[0]
    nsteps = L - 1
    gate = sc_ref[0]
    pe_scale = sc_ref[1]
    ident = jnp.where(
        lax.broadcasted_iota(jnp.int32, (_D, _D), 0)
        == lax.broadcasted_iota(jnp.int32, (_D, _D), 1), 1.0, 0.0)

    be = _extract32(bg_ref[...], eidx_ref[...] & 3)
    hrow = _extract32(hg_ref[...], ids_ref[...] & 3)
    h = (hrow + pe_scale * pe_ref[...]) * a_ref[...]
    cval = be[0:nsteps, :] + h[1:, :]
    c_ref[pl.ds(0, nsteps), :] = cval
    ccol_ref[pl.ds(0, nsteps)] = jnp.sum(
        ident[None, :, :] * cval[:, None, :], axis=2, keepdims=True)
    w3t_ref[...] = lax.dot_general(W3_ref[...], ident,
                                   (((1,), (0,)), ((), ())),
                                   preferred_element_type=jnp.float32)
    e0 = h[0:1, :]

    def stepA(t, e_row):
        E = jnp.broadcast_to(e_row, (_D, _D))
        r = jnp.sum(W3_ref[t] * E, axis=1, keepdims=True)
        ec = jnp.sum(ident * E, axis=1, keepdims=True)
        en = jnp.tanh(r + ccol_ref[t])
        return gate * en + (1.0 - gate) * ec

    def stepB(t, e_col):
        E = jnp.broadcast_to(e_col, (_D, _D))
        r = jnp.sum(w3t_ref[t] * E, axis=0, keepdims=True)
        er = jnp.sum(ident * E, axis=0, keepdims=True)
        en = jnp.tanh(r + c_ref[pl.ds(t, 1), :])
        return gate * en + (1.0 - gate) * er

    def pair(p, e_row):
        return stepB(2 * p + 1, stepA(2 * p, e_row))

    e = lax.fori_loop(0, (nsteps - 1) // 2, pair, e0)
    ecol = stepA(nsteps - 1, e)
    E = jnp.broadcast_to(ecol, (_D, _D))
    e = jnp.sum(ident * E, axis=0, keepdims=True)

    logits = lax.dot_general(e, bt_ref[...], (((1,), (1,)), ((), ())),
                             preferred_element_type=jnp.float32)
    logits_ref[...] = logits
    m = jnp.max(logits, axis=1, keepdims=True)
    ex = jnp.exp(logits - m)
    probs_ref[...] = ex / jnp.sum(ex, axis=1, keepdims=True)


def kernel(ids, eids, bias_table, W, bias, W_shared, bias_shared, a, gate,
           pe_scale, PE_cache):
    L = ids.shape[0]
    E = W.shape[0]
    rpw = L // _NW
    Wf = W.reshape(E, _D * _D)
    bias4 = bias.reshape(E // 4, 4 * _D)
    bt4 = bias_table.reshape(_V // 4, 4 * _D)
    eidx = jnp.concatenate([eids, eids[:1]]).astype(jnp.int32)
    ids32 = ids.astype(jnp.int32)
    eg4 = eidx // 4
    idg4 = ids32 // 4

    sc_gather = pl.kernel(
        _sc_gather_body,
        out_type=[jax.ShapeDtypeStruct((L, _D * _D), jnp.float32),
                  jax.ShapeDtypeStruct((L, 4 * _D), jnp.float32),
                  jax.ShapeDtypeStruct((L, 4 * _D), jnp.float32)],
        mesh=plsc.VectorSubcoreMesh(core_axis_name="c", subcore_axis_name="s"),
        scratch_types=[pltpu.VMEM((rpw,), jnp.int32),
                       pltpu.VMEM((rpw,), jnp.int32),
                       pltpu.VMEM((rpw,), jnp.int32),
                       pltpu.VMEM((rpw, _D * _D), jnp.float32),
                       pltpu.VMEM((rpw, 4 * _D), jnp.float32),
                       pltpu.VMEM((rpw, 4 * _D), jnp.float32),
                       pltpu.SemaphoreType.DMA,
                       pltpu.SemaphoreType.DMA,
                       pltpu.SemaphoreType.DMA],
    )
    Wg, bg, hg = sc_gather(Wf, bias4, bt4, eidx, eg4, idg4)
    W3 = Wg.reshape(L, _D, _D)

    sc2 = jnp.stack([jnp.asarray(gate, jnp.float32),
                     jnp.asarray(pe_scale, jnp.float32)])
    a2d = a[0].astype(jnp.float32)
    eidx2 = eidx.reshape(L, 1)
    ids2 = ids32.reshape(L, 1)

    logits2, probs2 = pl.pallas_call(
        _tc_scan_body,
        out_shape=[jax.ShapeDtypeStruct((1, _V), jnp.float32),
                   jax.ShapeDtypeStruct((1, _V), jnp.float32)],
        in_specs=[pl.BlockSpec(memory_space=pltpu.VMEM)] * 8
        + [pl.BlockSpec(memory_space=pltpu.SMEM)],
        out_specs=[pl.BlockSpec(memory_space=pltpu.VMEM)] * 2,
        scratch_shapes=[pltpu.VMEM((L, _D), jnp.float32),
                        pltpu.VMEM((L, _D, 1), jnp.float32),
                        pltpu.VMEM((L, _D, _D), jnp.float32)],
    )(W3, bg, hg, eidx2, ids2, PE_cache, a2d, bias_table, sc2)
    return logits2[0], probs2[0]


# speedup vs baseline: 2.5872x; 1.0136x over previous
import jax
import jax.numpy as jnp
from jax import lax
from jax.experimental import pallas as pl
from jax.experimental.pallas import tpu as pltpu
from jax.experimental.pallas import tpu_sc as plsc

_V = 4096
_D = 32
_NC = 2
_NS = 16
_NW = _NC * _NS


def _sc_gather_body(Wf_hbm, bias4_hbm, bt4_hbm, eidx_hbm, eg4_hbm, idg4_hbm,
                    Wout_hbm, bout_hbm, hout_hbm,
                    eidx_v, eg4_v, idg4_v, wrows_v, brows_v, hrows_v,
                    sem_w, sem_b, sem_h):
    rpw = eidx_v.shape[0]
    wid = lax.axis_index("s") * _NC + lax.axis_index("c")
    base = wid * rpw
    pltpu.sync_copy(eidx_hbm.at[pl.ds(base, rpw)], eidx_v)
    pltpu.sync_copy(eg4_hbm.at[pl.ds(base, rpw)], eg4_v)
    pltpu.sync_copy(idg4_hbm.at[pl.ds(base, rpw)], idg4_v)
    cw = pltpu.async_copy(Wf_hbm.at[eidx_v], wrows_v, sem_w)
    cb = pltpu.async_copy(bias4_hbm.at[eg4_v], brows_v, sem_b)
    ch = pltpu.async_copy(bt4_hbm.at[idg4_v], hrows_v, sem_h)
    cw.wait()
    cb.wait()
    ch.wait()
    pltpu.sync_copy(wrows_v, Wout_hbm.at[pl.ds(base, rpw)])
    pltpu.sync_copy(brows_v, bout_hbm.at[pl.ds(base, rpw)])
    pltpu.sync_copy(hrows_v, hout_hbm.at[pl.ds(base, rpw)])


def _extract32(rows128, sub):
    lane_grp = lax.broadcasted_iota(jnp.int32, (1, 128), 1) // _D
    masked = jnp.where(lane_grp == sub, rows128, 0.0)
    return (masked[:, 0:32] + masked[:, 32:64]
            + masked[:, 64:96] + masked[:, 96:128])


def _tc_scan_body(W3_ref, bg_ref, hg_ref, eidx_ref, ids_ref, pe_ref, a_ref,
                  bt_ref, sc_ref, logits_ref, probs_ref,
                  c_ref, ccol_ref, w3t_ref):
    L = hg_ref.shape[0]
    nsteps = L - 1
    gate = sc_ref[0]
    pe_scale = sc_ref[1]
    ident = jnp.where(
        lax.broadcasted_iota(jnp.int32, (_D, _D), 0)
        == lax.broadcasted_iota(jnp.int32, (_D, _D), 1), 1.0, 0.0)

    be = _extract32(bg_ref[...], eidx_ref[...] & 3)
    hrow = _extract32(hg_ref[...], ids_ref[...] & 3)
    h = (hrow + pe_scale * pe_ref[...]) * a_ref[...]
    cval = be[0:nsteps, :] + h[1:, :]
    c_ref[pl.ds(0, nsteps), :] = cval
    ccol_ref[pl.ds(0, nsteps)] = jnp.sum(
        ident[None, :, :] * cval[:, None, :], axis=2, keepdims=True)
    w3t_ref[...] = lax.dot_general(W3_ref[...], ident,
                                   (((1,), (0,)), ((), ())),
                                   preferred_element_type=jnp.float32)
    e0 = h[0:1, :]

    def stepA(t, e_row):
        E = jnp.broadcast_to(e_row, (_D, _D))
        r = jnp.sum(W3_ref[t] * E, axis=1, keepdims=True)
        ec = jnp.sum(ident * E, axis=1, keepdims=True)
        en = jnp.tanh(r + ccol_ref[t])
        return gate * en + (1.0 - gate) * ec

    def stepB(t, e_col):
        E = jnp.broadcast_to(e_col, (_D, _D))
        r = jnp.sum(w3t_ref[t] * E, axis=0, keepdims=True)
        er = jnp.sum(ident * E, axis=0, keepdims=True)
        en = jnp.tanh(r + c_ref[pl.ds(t, 1), :])
        return gate * en + (1.0 - gate) * er

    def pair(p, e_row):
        return stepB(2 * p + 1, stepA(2 * p, e_row))

    e = lax.fori_loop(0, (nsteps - 1) // 2, pair, e0, unroll=5)
    ecol = stepA(nsteps - 1, e)
    E = jnp.broadcast_to(ecol, (_D, _D))
    e = jnp.sum(ident * E, axis=0, keepdims=True)

    logits = lax.dot_general(e, bt_ref[...], (((1,), (1,)), ((), ())),
                             preferred_element_type=jnp.float32)
    logits_ref[...] = logits
    m = jnp.max(logits, axis=1, keepdims=True)
    ex = jnp.exp(logits - m)
    probs_ref[...] = ex / jnp.sum(ex, axis=1, keepdims=True)


def kernel(ids, eids, bias_table, W, bias, W_shared, bias_shared, a, gate,
           pe_scale, PE_cache):
    L = ids.shape[0]
    E = W.shape[0]
    rpw = L // _NW
    Wf = W.reshape(E, _D * _D)
    bias4 = bias.reshape(E // 4, 4 * _D)
    bt4 = bias_table.reshape(_V // 4, 4 * _D)
    eidx = jnp.concatenate([eids, eids[:1]]).astype(jnp.int32)
    ids32 = ids.astype(jnp.int32)
    eg4 = eidx // 4
    idg4 = ids32 // 4

    sc_gather = pl.kernel(
        _sc_gather_body,
        out_type=[jax.ShapeDtypeStruct((L, _D * _D), jnp.float32),
                  jax.ShapeDtypeStruct((L, 4 * _D), jnp.float32),
                  jax.ShapeDtypeStruct((L, 4 * _D), jnp.float32)],
        mesh=plsc.VectorSubcoreMesh(core_axis_name="c", subcore_axis_name="s"),
        scratch_types=[pltpu.VMEM((rpw,), jnp.int32),
                       pltpu.VMEM((rpw,), jnp.int32),
                       pltpu.VMEM((rpw,), jnp.int32),
                       pltpu.VMEM((rpw, _D * _D), jnp.float32),
                       pltpu.VMEM((rpw, 4 * _D), jnp.float32),
                       pltpu.VMEM((rpw, 4 * _D), jnp.float32),
                       pltpu.SemaphoreType.DMA,
                       pltpu.SemaphoreType.DMA,
                       pltpu.SemaphoreType.DMA],
    )
    Wg, bg, hg = sc_gather(Wf, bias4, bt4, eidx, eg4, idg4)
    W3 = Wg.reshape(L, _D, _D)

    sc2 = jnp.stack([jnp.asarray(gate, jnp.float32),
                     jnp.asarray(pe_scale, jnp.float32)])
    a2d = a[0].astype(jnp.float32)
    eidx2 = eidx.reshape(L, 1)
    ids2 = ids32.reshape(L, 1)

    logits2, probs2 = pl.pallas_call(
        _tc_scan_body,
        out_shape=[jax.ShapeDtypeStruct((1, _V), jnp.float32),
                   jax.ShapeDtypeStruct((1, _V), jnp.float32)],
        in_specs=[pl.BlockSpec(memory_space=pltpu.VMEM)] * 8
        + [pl.BlockSpec(memory_space=pltpu.SMEM)],
        out_specs=[pl.BlockSpec(memory_space=pltpu.VMEM)] * 2,
        scratch_shapes=[pltpu.VMEM((L, _D), jnp.float32),
                        pltpu.VMEM((L, _D, 1), jnp.float32),
                        pltpu.VMEM((L, _D, _D), jnp.float32)],
    )(W3, bg, hg, eidx2, ids2, PE_cache, a2d, bias_table, sc2)
    return logits2[0], probs2[0]


# speedup vs baseline: 2.6055x; 1.0071x over previous
import jax
import jax.numpy as jnp
from jax import lax
from jax.experimental import pallas as pl
from jax.experimental.pallas import tpu as pltpu
from jax.experimental.pallas import tpu_sc as plsc

_V = 4096
_D = 32
_NC = 2
_NS = 16
_NW = _NC * _NS


def _sc_gather_body(Wf_hbm, bias4_hbm, eidx_hbm, eg4_hbm,
                    Wout_hbm, bout_hbm,
                    eidx_v, eg4_v, wrows_v, brows_v,
                    sem_w, sem_b):
    rpw = eidx_v.shape[0]
    wid = lax.axis_index("s") * _NC + lax.axis_index("c")
    base = wid * rpw
    pltpu.sync_copy(eidx_hbm.at[pl.ds(base, rpw)], eidx_v)
    pltpu.sync_copy(eg4_hbm.at[pl.ds(base, rpw)], eg4_v)
    cw = pltpu.async_copy(Wf_hbm.at[eidx_v], wrows_v, sem_w)
    cb = pltpu.async_copy(bias4_hbm.at[eg4_v], brows_v, sem_b)
    cw.wait()
    cb.wait()
    pltpu.sync_copy(wrows_v, Wout_hbm.at[pl.ds(base, rpw)])
    pltpu.sync_copy(brows_v, bout_hbm.at[pl.ds(base, rpw)])


def _extract32(rows128, sub):
    lane_grp = lax.broadcasted_iota(jnp.int32, (1, 128), 1) // _D
    masked = jnp.where(lane_grp == sub, rows128, 0.0)
    return (masked[:, 0:32] + masked[:, 32:64]
            + masked[:, 64:96] + masked[:, 96:128])


def _tc_scan_body(W3_ref, bg_ref, eidx_ref, ids_ref, pe_ref, a_ref,
                  bt_ref, sc_ref, logits_ref, probs_ref,
                  c_ref, ccol_ref, w3t_ref):
    L = ids_ref.shape[0]
    nsteps = L - 1
    gate = sc_ref[0]
    pe_scale = sc_ref[1]
    ident = jnp.where(
        lax.broadcasted_iota(jnp.int32, (_D, _D), 0)
        == lax.broadcasted_iota(jnp.int32, (_D, _D), 1), 1.0, 0.0)

    be = _extract32(bg_ref[...], eidx_ref[...] & 3)
    onehot = jnp.where(
        ids_ref[...] == lax.broadcasted_iota(jnp.int32, (1, _V), 1), 1.0, 0.0)
    hrow = lax.dot_general(onehot, bt_ref[...], (((1,), (0,)), ((), ())),
                           preferred_element_type=jnp.float32)
    h = (hrow + pe_scale * pe_ref[...]) * a_ref[...]
    cval = be[0:nsteps, :] + h[1:, :]
    c_ref[pl.ds(0, nsteps), :] = cval
    ccol_ref[pl.ds(0, nsteps)] = jnp.sum(
        ident[None, :, :] * cval[:, None, :], axis=2, keepdims=True)
    w3t_ref[...] = lax.dot_general(W3_ref[...], ident,
                                   (((1,), (0,)), ((), ())),
                                   preferred_element_type=jnp.float32)
    e0 = h[0:1, :]

    def stepA(t, e_row):
        E = jnp.broadcast_to(e_row, (_D, _D))
        r = jnp.sum(W3_ref[t] * E, axis=1, keepdims=True)
        ec = jnp.sum(ident * E, axis=1, keepdims=True)
        en = jnp.tanh(r + ccol_ref[t])
        return gate * en + (1.0 - gate) * ec

    def stepB(t, e_col):
        E = jnp.broadcast_to(e_col, (_D, _D))
        r = jnp.sum(w3t_ref[t] * E, axis=0, keepdims=True)
        er = jnp.sum(ident * E, axis=0, keepdims=True)
        en = jnp.tanh(r + c_ref[pl.ds(t, 1), :])
        return gate * en + (1.0 - gate) * er

    def pair(p, e_row):
        return stepB(2 * p + 1, stepA(2 * p, e_row))

    e = lax.fori_loop(0, (nsteps - 1) // 2, pair, e0, unroll=5)
    ecol = stepA(nsteps - 1, e)
    E = jnp.broadcast_to(ecol, (_D, _D))
    e = jnp.sum(ident * E, axis=0, keepdims=True)

    logits = lax.dot_general(e, bt_ref[...], (((1,), (1,)), ((), ())),
                             preferred_element_type=jnp.float32)
    logits_ref[...] = logits
    m = jnp.max(logits, axis=1, keepdims=True)
    ex = jnp.exp(logits - m)
    probs_ref[...] = ex / jnp.sum(ex, axis=1, keepdims=True)


def kernel(ids, eids, bias_table, W, bias, W_shared, bias_shared, a, gate,
           pe_scale, PE_cache):
    L = ids.shape[0]
    E = W.shape[0]
    rpw = L // _NW
    Wf = W.reshape(E, _D * _D)
    bias4 = bias.reshape(E // 4, 4 * _D)
    eidx = jnp.concatenate([eids, eids[:1]]).astype(jnp.int32)
    ids32 = ids.astype(jnp.int32)
    eg4 = eidx // 4

    sc_gather = pl.kernel(
        _sc_gather_body,
        out_type=[jax.ShapeDtypeStruct((L, _D * _D), jnp.float32),
                  jax.ShapeDtypeStruct((L, 4 * _D), jnp.float32)],
        mesh=plsc.VectorSubcoreMesh(core_axis_name="c", subcore_axis_name="s"),
        scratch_types=[pltpu.VMEM((rpw,), jnp.int32),
                       pltpu.VMEM((rpw,), jnp.int32),
                       pltpu.VMEM((rpw, _D * _D), jnp.float32),
                       pltpu.VMEM((rpw, 4 * _D), jnp.float32),
                       pltpu.SemaphoreType.DMA,
                       pltpu.SemaphoreType.DMA],
    )
    Wg, bg = sc_gather(Wf, bias4, eidx, eg4)
    W3 = Wg.reshape(L, _D, _D)

    sc2 = jnp.stack([jnp.asarray(gate, jnp.float32),
                     jnp.asarray(pe_scale, jnp.float32)])
    a2d = a[0].astype(jnp.float32)
    eidx2 = eidx.reshape(L, 1)
    ids2 = ids32.reshape(L, 1)

    logits2, probs2 = pl.pallas_call(
        _tc_scan_body,
        out_shape=[jax.ShapeDtypeStruct((1, _V), jnp.float32),
                   jax.ShapeDtypeStruct((1, _V), jnp.float32)],
        in_specs=[pl.BlockSpec(memory_space=pltpu.VMEM)] * 7
        + [pl.BlockSpec(memory_space=pltpu.SMEM)],
        out_specs=[pl.BlockSpec(memory_space=pltpu.VMEM)] * 2,
        scratch_shapes=[pltpu.VMEM((L, _D), jnp.float32),
                        pltpu.VMEM((L, _D, 1), jnp.float32),
                        pltpu.VMEM((L, _D, _D), jnp.float32)],
    )(W3, bg, eidx2, ids2, PE_cache, a2d, bias_table, sc2)
    return logits2[0], probs2[0]
